# Initial kernel scaffold; baseline (speedup 1.0000x reference)
import jax
import jax.numpy as jnp
from jax.experimental import pallas as pl


def kernel(h, x, edge_index, fW1, fb1, fW2, fb2, uW1, ub1, uW2, ub2):
    raise NotImplementedError("write your pallas kernel here")


# speedup vs baseline: 3.5698x; 3.5698x over previous
import functools

import jax
import jax.numpy as jnp
from jax import lax
from jax.experimental import pallas as pl
from jax.experimental.pallas import tpu as pltpu
from jax.experimental.pallas import tpu_sc as plsc

NC = 2
NS = 16
L = 16
NW = NC * NS

H = 128
R = 32
N = 10000
NPAD = 10112
RPT = NPAD // NS
DUMMY = N + 8

E = 320000
CH = 128
EPT = ((E // NW) + CH - 1) // CH * CH
NCHUNK = EPT // CH
EPAD = EPT * NW
BE = 1024
NBLK = EPAD // BE
BN = 1000
GAMMA = 10.0


def _ssp(v):
    return jnp.maximum(v, 0.0) + jnp.log1p(jnp.exp(-jnp.abs(v))) - 0.5


def _geom_body(x8_hbm, ii_hbm, jj_hbm, ones_hbm, z1_hbm,
               s_out, deg_out,
               xbuf, ibuf, jbuf, sbuf, onesbuf, zbuf, deg_sp):
    cid = lax.axis_index("c")
    sid = lax.axis_index("s")
    wid = cid * NS + sid
    r0 = sid * RPT

    pltpu.sync_copy(x8_hbm, xbuf)
    pltpu.sync_copy(ones_hbm, onesbuf)
    pltpu.sync_copy(z1_hbm.at[pl.ds(r0, RPT)], zbuf)
    pltpu.sync_copy(zbuf, deg_sp.at[pl.ds(r0, RPT)])
    plsc.subcore_barrier()

    base = wid * EPT

    def chunk(k, carry):
        off = base + k * CH
        pltpu.sync_copy(ii_hbm.at[pl.ds(off, CH)], ibuf)
        pltpu.sync_copy(jj_hbm.at[pl.ds(off, CH)], jbuf)
        for g in range(CH // L):
            ri = ibuf[pl.ds(g * L, L)] * 8
            rj = jbuf[pl.ds(g * L, L)] * 8
            s_acc = None
            for c in range(3):
                cc = jnp.full((L,), c, jnp.int32)
                a = plsc.load_gather(xbuf, [ri + cc])
                b = plsc.load_gather(xbuf, [rj + cc])
                d = a - b
                s_acc = d * d if s_acc is None else s_acc + d * d
            sbuf[pl.ds(g * L, L)] = s_acc
        pltpu.sync_copy(sbuf, s_out.at[pl.ds(off, CH)])
        pltpu.sync_copy(onesbuf, deg_sp.at[ibuf], add=True)
        return carry

    lax.fori_loop(0, NCHUNK, chunk, 0)
    plsc.subcore_barrier()

    pltpu.sync_copy(deg_sp.at[pl.ds(r0, RPT)], zbuf)
    pltpu.sync_copy(zbuf, deg_out.at[pl.ds(cid * NPAD + r0, RPT)])


def _geom(x8, ii, jj, ones1, z1):
    mesh = plsc.VectorSubcoreMesh(core_axis_name="c", subcore_axis_name="s",
                                  num_cores=NC, num_subcores=NS)
    return pl.kernel(
        _geom_body,
        out_type=(jax.ShapeDtypeStruct((EPAD,), jnp.float32),
                  jax.ShapeDtypeStruct((NC * NPAD,), jnp.float32)),
        mesh=mesh,
        compiler_params=pltpu.CompilerParams(needs_layout_passes=False),
        scratch_types=[
            pltpu.VMEM((NPAD * 8,), jnp.float32),
            pltpu.VMEM((CH,), jnp.int32),
            pltpu.VMEM((CH,), jnp.int32),
            pltpu.VMEM((CH,), jnp.float32),
            pltpu.VMEM((CH,), jnp.float32),
            pltpu.VMEM((RPT,), jnp.float32),
            pltpu.VMEM_SHARED((NPAD,), jnp.float32),
        ],
    )(x8, ii, jj, ones1, z1)


def _filter_body(s_ref, fW1_ref, fb1_ref, fW2_ref, fb2_ref, w_ref):
    srow = s_ref[0]
    d = jnp.sqrt(srow + 1e-12)
    mu = lax.broadcasted_iota(jnp.int32, (R, 1), 0).astype(jnp.float32) * (
        5.0 / (R - 1))
    rbf_t = jnp.exp(-GAMMA * (d - mu) ** 2)
    a = lax.dot_general(fW1_ref[...], rbf_t, (((0,), (0,)), ((), ())),
                        preferred_element_type=jnp.float32)
    z = _ssp(a + fb1_ref[...])
    w = lax.dot_general(z, fW2_ref[...], (((0,), (0,)), ((), ())),
                        preferred_element_type=jnp.float32)
    w_ref[...] = w + fb2_ref[...]


def _filter(s, fW1, fb1, fW2, fb2):
    s3 = s.reshape(NBLK, 1, BE)
    return pl.pallas_call(
        _filter_body,
        grid=(NBLK,),
        in_specs=[
            pl.BlockSpec((1, 1, BE), lambda i: (i, 0, 0)),
            pl.BlockSpec((R, H), lambda i: (0, 0)),
            pl.BlockSpec((H, 1), lambda i: (0, 0)),
            pl.BlockSpec((H, H), lambda i: (0, 0)),
            pl.BlockSpec((1, H), lambda i: (0, 0)),
        ],
        out_specs=pl.BlockSpec((BE, H), lambda i: (i, 0)),
        out_shape=jax.ShapeDtypeStruct((EPAD, H), jnp.float32),
    )(s3, fW1, fb1.reshape(H, 1), fW2, fb2.reshape(1, H))


def _msg_body(w_hbm, h_hbm, ii_hbm, jj_hbm, z128_hbm,
              aggr_out,
              ibuf, jbuf, wbuf, hbuf, zbuf, aggr_sp, gsem):
    cid = lax.axis_index("c")
    sid = lax.axis_index("s")
    wid = cid * NS + sid
    r0 = sid * RPT

    for tt in range(8):
        zr = 80 if tt < 7 else 72
        sl = pl.ds(r0 + tt * 80, zr)
        pltpu.sync_copy(z128_hbm.at[pl.ds(tt * 80, zr)], zbuf.at[pl.ds(0, zr)])
        pltpu.sync_copy(zbuf.at[pl.ds(0, zr)], aggr_sp.at[sl])
    plsc.subcore_barrier()

    base = wid * EPT

    def chunk(k, carry):
        off = base + k * CH
        pltpu.sync_copy(ii_hbm.at[pl.ds(off, CH)], ibuf)
        pltpu.sync_copy(jj_hbm.at[pl.ds(off, CH)], jbuf)
        pltpu.sync_copy(w_hbm.at[pl.ds(off, CH)], wbuf)
        pltpu.async_copy(h_hbm.at[jbuf], hbuf, gsem).wait()

        def emul(e, c2):
            for c in range(H // L):
                sl = pl.ds(c * L, L)
                wbuf[e, sl] = wbuf[e, sl] * hbuf[e, sl]
            return c2

        lax.fori_loop(0, CH, emul, 0)
        pltpu.sync_copy(wbuf, aggr_sp.at[ibuf], add=True)
        return carry

    lax.fori_loop(0, NCHUNK, chunk, 0)
    plsc.subcore_barrier()

    for tt in range(8):
        zr = 80 if tt < 7 else 72
        sl = pl.ds(r0 + tt * 80, zr)
        pltpu.sync_copy(aggr_sp.at[sl], zbuf.at[pl.ds(0, zr)])
        pltpu.sync_copy(zbuf.at[pl.ds(0, zr)], aggr_out.at[cid, pl.ds(r0 + tt * 80, zr)])


def _msg(w, h, ii, jj, z128):
    mesh = plsc.VectorSubcoreMesh(core_axis_name="c", subcore_axis_name="s",
                                  num_cores=NC, num_subcores=NS)
    return pl.kernel(
        _msg_body,
        out_type=jax.ShapeDtypeStruct((NC, NPAD, H), jnp.float32),
        mesh=mesh,
        compiler_params=pltpu.CompilerParams(needs_layout_passes=False),
        scratch_types=[
            pltpu.VMEM((CH,), jnp.int32),
            pltpu.VMEM((CH,), jnp.int32),
            pltpu.VMEM((CH, H), jnp.float32),
            pltpu.VMEM((CH, H), jnp.float32),
            pltpu.VMEM((80, H), jnp.float32),
            pltpu.VMEM_SHARED((NPAD, H), jnp.float32),
            pltpu.SemaphoreType.DMA,
        ],
    )(w, h, ii, jj, z128)


def _update_body(aggr_ref, deg_ref, uW1_ref, ub1_ref, uW2_ref, ub2_ref, out_ref):
    a = aggr_ref[0] + aggr_ref[1]
    dg = deg_ref[0] + deg_ref[1]
    a = a / jnp.maximum(dg, 1.0)
    z = _ssp(jnp.dot(a, uW1_ref[...],
                     preferred_element_type=jnp.float32) + ub1_ref[...])
    out_ref[...] = jnp.dot(z, uW2_ref[...],
                           preferred_element_type=jnp.float32) + ub2_ref[...]


def _update(aggr, deg, uW1, ub1, uW2, ub2):
    return pl.pallas_call(
        _update_body,
        grid=(N // BN,),
        in_specs=[
            pl.BlockSpec((NC, BN, H), lambda i: (0, i, 0)),
            pl.BlockSpec((NC, BN, 1), lambda i: (0, i, 0)),
            pl.BlockSpec((H, H), lambda i: (0, 0)),
            pl.BlockSpec((1, H), lambda i: (0, 0)),
            pl.BlockSpec((H, H), lambda i: (0, 0)),
            pl.BlockSpec((1, H), lambda i: (0, 0)),
        ],
        out_specs=pl.BlockSpec((BN, H), lambda i: (i, 0)),
        out_shape=jax.ShapeDtypeStruct((N, H), jnp.float32),
    )(aggr, deg, uW1, ub1.reshape(1, H), uW2, ub2.reshape(1, H))


def kernel(h, x, edge_index, fW1, fb1, fW2, fb2, uW1, ub1, uW2, ub2):
    ii = edge_index[0].astype(jnp.int32)
    jj = edge_index[1].astype(jnp.int32)
    npad = EPAD - E
    ii = jnp.concatenate([ii, jnp.full((npad,), DUMMY, jnp.int32)])
    jj = jnp.concatenate([jj, jnp.zeros((npad,), jnp.int32)])

    x8 = jnp.zeros((NPAD, 8), jnp.float32).at[:N, :3].set(x).reshape(NPAD * 8)
    z1 = jnp.zeros((NPAD,), jnp.float32)
    z128 = jnp.zeros((NPAD, H), jnp.float32)
    ones1 = jnp.ones((CH,), jnp.float32)

    s, deg = _geom(x8, ii, jj, ones1, z1)
    w = _filter(s, fW1, fb1, fW2, fb2)
    aggr = _msg(w, h, ii, jj, z128)
    h_new = _update(aggr, deg.reshape(NC, NPAD, 1), uW1, ub1, uW2, ub2)
    return (h_new, x)


# speedup vs baseline: 4.1842x; 1.1721x over previous
import functools

import jax
import jax.numpy as jnp
from jax import lax
from jax.experimental import pallas as pl
from jax.experimental.pallas import tpu as pltpu
from jax.experimental.pallas import tpu_sc as plsc

NC = 2
NS = 16
L = 16
NW = NC * NS

H = 128
R = 32
N = 10000
NPAD = 10112
RPT = NPAD // NS
DUMMY = N + 8

E = 320000
CH = 128
EPT = ((E // NW) + CH - 1) // CH * CH
NCHUNK = EPT // CH
EPAD = EPT * NW
BE = 1024
NBLK = EPAD // BE
BN = 1000
GAMMA = 10.0
MCH = 64
MNCHUNK = EPT // MCH


def _ssp(v):
    return jnp.maximum(v, 0.0) + jnp.log1p(jnp.exp(-jnp.abs(v))) - 0.5


def _geom_body(x8_hbm, ii_hbm, jj_hbm, ones_hbm, z1_hbm,
               s_out, deg_out,
               xbuf, ibuf, jbuf, sbuf, onesbuf, zbuf, deg_sp):
    cid = lax.axis_index("c")
    sid = lax.axis_index("s")
    wid = cid * NS + sid
    r0 = sid * RPT

    pltpu.sync_copy(x8_hbm, xbuf)
    pltpu.sync_copy(ones_hbm, onesbuf)
    pltpu.sync_copy(z1_hbm.at[pl.ds(r0, RPT)], zbuf)
    pltpu.sync_copy(zbuf, deg_sp.at[pl.ds(r0, RPT)])
    plsc.subcore_barrier()

    base = wid * EPT

    def chunk(k, carry):
        off = base + k * CH
        pltpu.sync_copy(ii_hbm.at[pl.ds(off, CH)], ibuf)
        pltpu.sync_copy(jj_hbm.at[pl.ds(off, CH)], jbuf)
        for g in range(CH // L):
            ri = ibuf[pl.ds(g * L, L)] * 8
            rj = jbuf[pl.ds(g * L, L)] * 8
            s_acc = None
            for c in range(3):
                cc = jnp.full((L,), c, jnp.int32)
                a = plsc.load_gather(xbuf, [ri + cc])
                b = plsc.load_gather(xbuf, [rj + cc])
                d = a - b
                s_acc = d * d if s_acc is None else s_acc + d * d
            sbuf[pl.ds(g * L, L)] = s_acc
        pltpu.sync_copy(sbuf, s_out.at[pl.ds(off, CH)])
        pltpu.sync_copy(onesbuf, deg_sp.at[ibuf], add=True)
        return carry

    lax.fori_loop(0, NCHUNK, chunk, 0)
    plsc.subcore_barrier()

    pltpu.sync_copy(deg_sp.at[pl.ds(r0, RPT)], zbuf)
    pltpu.sync_copy(zbuf, deg_out.at[pl.ds(cid * NPAD + r0, RPT)])


def _geom(x8, ii, jj, ones1, z1):
    mesh = plsc.VectorSubcoreMesh(core_axis_name="c", subcore_axis_name="s",
                                  num_cores=NC, num_subcores=NS)
    return pl.kernel(
        _geom_body,
        out_type=(jax.ShapeDtypeStruct((EPAD,), jnp.float32),
                  jax.ShapeDtypeStruct((NC * NPAD,), jnp.float32)),
        mesh=mesh,
        compiler_params=pltpu.CompilerParams(needs_layout_passes=False),
        scratch_types=[
            pltpu.VMEM((NPAD * 8,), jnp.float32),
            pltpu.VMEM((CH,), jnp.int32),
            pltpu.VMEM((CH,), jnp.int32),
            pltpu.VMEM((CH,), jnp.float32),
            pltpu.VMEM((CH,), jnp.float32),
            pltpu.VMEM((RPT,), jnp.float32),
            pltpu.VMEM_SHARED((NPAD,), jnp.float32),
        ],
    )(x8, ii, jj, ones1, z1)


def _filter_body(s_ref, fW1_ref, fb1_ref, fW2_ref, fb2_ref, w_ref):
    srow = s_ref[0]
    d = jnp.sqrt(srow + 1e-12)
    mu = lax.broadcasted_iota(jnp.int32, (R, 1), 0).astype(jnp.float32) * (
        5.0 / (R - 1))
    rbf_t = jnp.exp(-GAMMA * (d - mu) ** 2)
    a = lax.dot_general(fW1_ref[...], rbf_t, (((0,), (0,)), ((), ())),
                        preferred_element_type=jnp.float32)
    z = _ssp(a + fb1_ref[...])
    w = lax.dot_general(z, fW2_ref[...], (((0,), (0,)), ((), ())),
                        preferred_element_type=jnp.float32)
    w_ref[...] = w + fb2_ref[...]


def _filter(s, fW1, fb1, fW2, fb2):
    s3 = s.reshape(NBLK, 1, BE)
    return pl.pallas_call(
        _filter_body,
        grid=(NBLK,),
        in_specs=[
            pl.BlockSpec((1, 1, BE), lambda i: (i, 0, 0)),
            pl.BlockSpec((R, H), lambda i: (0, 0)),
            pl.BlockSpec((H, 1), lambda i: (0, 0)),
            pl.BlockSpec((H, H), lambda i: (0, 0)),
            pl.BlockSpec((1, H), lambda i: (0, 0)),
        ],
        out_specs=pl.BlockSpec((BE, H), lambda i: (i, 0)),
        out_shape=jax.ShapeDtypeStruct((EPAD, H), jnp.float32),
    )(s3, fW1, fb1.reshape(H, 1), fW2, fb2.reshape(1, H))


def _msg_body(w_hbm, h_hbm, ii_hbm, jj_hbm, z128_hbm,
              aggr_out,
              ibuf0, ibuf1, jbuf0, jbuf1, wbuf0, wbuf1, hbuf0, hbuf1, zbuf,
              aggr_sp,
              sii0, sii1, sjj0, sjj1, sw0, sw1, sg0, sg1, ss0, ss1):
    cid = lax.axis_index("c")
    sid = lax.axis_index("s")
    wid = cid * NS + sid
    r0 = sid * RPT

    ibuf = (ibuf0, ibuf1)
    jbuf = (jbuf0, jbuf1)
    wbuf = (wbuf0, wbuf1)
    hbuf = (hbuf0, hbuf1)
    sii = (sii0, sii1)
    sjj = (sjj0, sjj1)
    sw = (sw0, sw1)
    sg = (sg0, sg1)
    ss = (ss0, ss1)

    for tt in range(8):
        zr = 80 if tt < 7 else 72
        sl = pl.ds(r0 + tt * 80, zr)
        pltpu.sync_copy(z128_hbm.at[pl.ds(tt * 80, zr)], zbuf.at[pl.ds(0, zr)])
        pltpu.sync_copy(zbuf.at[pl.ds(0, zr)], aggr_sp.at[sl])
    plsc.subcore_barrier()

    base = wid * EPT

    def start_inputs(k, b):
        off = base + k * MCH
        pltpu.async_copy(ii_hbm.at[pl.ds(off, MCH)], ibuf[b], sii[b])
        pltpu.async_copy(jj_hbm.at[pl.ds(off, MCH)], jbuf[b], sjj[b])
        pltpu.async_copy(w_hbm.at[pl.ds(off, MCH)], wbuf[b], sw[b])

    def wait_idx(k, b):
        off = base + k * MCH
        pltpu.make_async_copy(ii_hbm.at[pl.ds(off, MCH)], ibuf[b], sii[b]).wait()
        pltpu.make_async_copy(jj_hbm.at[pl.ds(off, MCH)], jbuf[b], sjj[b]).wait()

    def work(k, b, pref, swait):
        off = base + k * MCH
        if swait:
            pltpu.make_async_copy(
                wbuf[1 - b], aggr_sp.at[ibuf[1 - b]], ss[1 - b]).wait()
        if pref:
            start_inputs(k + 1, 1 - b)
        pltpu.make_async_copy(w_hbm.at[pl.ds(off, MCH)], wbuf[b], sw[b]).wait()
        pltpu.make_async_copy(h_hbm.at[jbuf[b]], hbuf[b], sg[b]).wait()

        def emul(e, c2):
            for c in range(H // L):
                sl = pl.ds(c * L, L)
                wbuf[b][e, sl] = wbuf[b][e, sl] * hbuf[b][e, sl]
            return c2

        lax.fori_loop(0, MCH, emul, 0)
        if pref:
            wait_idx(k + 1, 1 - b)
            pltpu.async_copy(h_hbm.at[jbuf[1 - b]], hbuf[1 - b], sg[1 - b])
        pltpu.async_copy(wbuf[b], aggr_sp.at[ibuf[b]], ss[b], add=True)

    start_inputs(0, 0)
    wait_idx(0, 0)
    pltpu.async_copy(h_hbm.at[jbuf[0]], hbuf[0], sg[0])
    work(0, 0, pref=True, swait=False)

    def pair(it, carry):
        k0 = 1 + it * 2
        work(k0, 1, True, True)
        work(k0 + 1, 0, True, True)
        return carry

    lax.fori_loop(0, (MNCHUNK - 2) // 2, pair, 0)
    work(MNCHUNK - 1, 1, pref=False, swait=True)
    pltpu.make_async_copy(wbuf[1], aggr_sp.at[ibuf[1]], ss[1]).wait()
    plsc.subcore_barrier()

    for tt in range(8):
        zr = 80 if tt < 7 else 72
        sl = pl.ds(r0 + tt * 80, zr)
        pltpu.sync_copy(aggr_sp.at[sl], zbuf.at[pl.ds(0, zr)])
        pltpu.sync_copy(zbuf.at[pl.ds(0, zr)], aggr_out.at[cid, pl.ds(r0 + tt * 80, zr)])


def _msg(w, h, ii, jj, z128):
    mesh = plsc.VectorSubcoreMesh(core_axis_name="c", subcore_axis_name="s",
                                  num_cores=NC, num_subcores=NS)
    return pl.kernel(
        _msg_body,
        out_type=jax.ShapeDtypeStruct((NC, NPAD, H), jnp.float32),
        mesh=mesh,
        compiler_params=pltpu.CompilerParams(needs_layout_passes=False),
        scratch_types=(
            [pltpu.VMEM((MCH,), jnp.int32)] * 4
            + [pltpu.VMEM((MCH, H), jnp.float32)] * 4
            + [pltpu.VMEM((80, H), jnp.float32),
               pltpu.VMEM_SHARED((NPAD, H), jnp.float32)]
            + [pltpu.SemaphoreType.DMA] * 10
        ),
    )(w, h, ii, jj, z128)


def _update_body(aggr_ref, deg_ref, uW1_ref, ub1_ref, uW2_ref, ub2_ref, out_ref):
    a = aggr_ref[0] + aggr_ref[1]
    dg = deg_ref[0] + deg_ref[1]
    a = a / jnp.maximum(dg, 1.0)
    z = _ssp(jnp.dot(a, uW1_ref[...],
                     preferred_element_type=jnp.float32) + ub1_ref[...])
    out_ref[...] = jnp.dot(z, uW2_ref[...],
                           preferred_element_type=jnp.float32) + ub2_ref[...]


def _update(aggr, deg, uW1, ub1, uW2, ub2):
    return pl.pallas_call(
        _update_body,
        grid=(N // BN,),
        in_specs=[
            pl.BlockSpec((NC, BN, H), lambda i: (0, i, 0)),
            pl.BlockSpec((NC, BN, 1), lambda i: (0, i, 0)),
            pl.BlockSpec((H, H), lambda i: (0, 0)),
            pl.BlockSpec((1, H), lambda i: (0, 0)),
            pl.BlockSpec((H, H), lambda i: (0, 0)),
            pl.BlockSpec((1, H), lambda i: (0, 0)),
        ],
        out_specs=pl.BlockSpec((BN, H), lambda i: (i, 0)),
        out_shape=jax.ShapeDtypeStruct((N, H), jnp.float32),
    )(aggr, deg, uW1, ub1.reshape(1, H), uW2, ub2.reshape(1, H))


def kernel(h, x, edge_index, fW1, fb1, fW2, fb2, uW1, ub1, uW2, ub2):
    ii = edge_index[0].astype(jnp.int32)
    jj = edge_index[1].astype(jnp.int32)
    npad = EPAD - E
    ii = jnp.concatenate([ii, jnp.full((npad,), DUMMY, jnp.int32)])
    jj = jnp.concatenate([jj, jnp.zeros((npad,), jnp.int32)])

    x8 = jnp.zeros((NPAD, 8), jnp.float32).at[:N, :3].set(x).reshape(NPAD * 8)
    z1 = jnp.zeros((NPAD,), jnp.float32)
    z128 = jnp.zeros((NPAD, H), jnp.float32)
    ones1 = jnp.ones((CH,), jnp.float32)

    s, deg = _geom(x8, ii, jj, ones1, z1)
    w = _filter(s, fW1, fb1, fW2, fb2)
    aggr = _msg(w, h, ii, jj, z128)
    h_new = _update(aggr, deg.reshape(NC, NPAD, 1), uW1, ub1, uW2, ub2)
    return (h_new, x)


# speedup vs baseline: 5.4368x; 1.2994x over previous
import functools

import jax
import jax.numpy as jnp
from jax import lax
from jax.experimental import pallas as pl
from jax.experimental.pallas import tpu as pltpu
from jax.experimental.pallas import tpu_sc as plsc

NC = 2
NS = 16
L = 16
NW = NC * NS

H = 128
R = 32
N = 10000
NPAD = 10112
RPT = NPAD // NS
DUMMY = N + 8

E = 320000
CH = 128
EPT = ((E // NW) + CH - 1) // CH * CH
NCHUNK = EPT // CH
EPAD = EPT * NW
BE = 1024
NBLK = EPAD // BE
BN = 1000
GAMMA = 10.0
MCH = 64
MNCHUNK = EPT // MCH
EPADH = EPAD // 2
EPTH = EPT // 2
MNCHUNKH = EPTH // MCH
NBLKH = EPADH // BE


def _ssp(v):
    return jnp.maximum(v, 0.0) + jnp.log1p(jnp.exp(-jnp.abs(v))) - 0.5


def _geom_body(x8_hbm, ii_hbm, jj_hbm, ones_hbm, z1_hbm,
               s_out, deg_out,
               xbuf, ibuf, jbuf, sbuf, onesbuf, zbuf, deg_sp):
    cid = lax.axis_index("c")
    sid = lax.axis_index("s")
    wid = cid * NS + sid
    r0 = sid * RPT

    pltpu.sync_copy(x8_hbm, xbuf)
    pltpu.sync_copy(ones_hbm, onesbuf)
    pltpu.sync_copy(z1_hbm.at[pl.ds(r0, RPT)], zbuf)
    pltpu.sync_copy(zbuf, deg_sp.at[pl.ds(r0, RPT)])
    plsc.subcore_barrier()

    base = wid * EPT

    def chunk(k, carry):
        off = base + k * CH
        pltpu.sync_copy(ii_hbm.at[pl.ds(off, CH)], ibuf)
        pltpu.sync_copy(jj_hbm.at[pl.ds(off, CH)], jbuf)
        for g in range(CH // L):
            ri = ibuf[pl.ds(g * L, L)] * 8
            rj = jbuf[pl.ds(g * L, L)] * 8
            s_acc = None
            for c in range(3):
                cc = jnp.full((L,), c, jnp.int32)
                a = plsc.load_gather(xbuf, [ri + cc])
                b = plsc.load_gather(xbuf, [rj + cc])
                d = a - b
                s_acc = d * d if s_acc is None else s_acc + d * d
            sbuf[pl.ds(g * L, L)] = s_acc
        pltpu.sync_copy(sbuf, s_out.at[pl.ds(off, CH)])
        pltpu.sync_copy(onesbuf, deg_sp.at[ibuf], add=True)
        return carry

    lax.fori_loop(0, NCHUNK, chunk, 0)
    plsc.subcore_barrier()

    pltpu.sync_copy(deg_sp.at[pl.ds(r0, RPT)], zbuf)
    pltpu.sync_copy(zbuf, deg_out.at[pl.ds(cid * NPAD + r0, RPT)])


def _geom(x8, ii, jj, ones1, z1):
    mesh = plsc.VectorSubcoreMesh(core_axis_name="c", subcore_axis_name="s",
                                  num_cores=NC, num_subcores=NS)
    return pl.kernel(
        _geom_body,
        out_type=(jax.ShapeDtypeStruct((EPAD,), jnp.float32),
                  jax.ShapeDtypeStruct((NC * NPAD,), jnp.float32)),
        mesh=mesh,
        compiler_params=pltpu.CompilerParams(needs_layout_passes=False),
        scratch_types=[
            pltpu.VMEM((NPAD * 8,), jnp.float32),
            pltpu.VMEM((CH,), jnp.int32),
            pltpu.VMEM((CH,), jnp.int32),
            pltpu.VMEM((CH,), jnp.float32),
            pltpu.VMEM((CH,), jnp.float32),
            pltpu.VMEM((RPT,), jnp.float32),
            pltpu.VMEM_SHARED((NPAD,), jnp.float32),
        ],
    )(x8, ii, jj, ones1, z1)


def _filter_body(s_ref, fW1_ref, fb1_ref, fW2_ref, fb2_ref, w_ref):
    srow = s_ref[0]
    d = jnp.sqrt(srow + 1e-12)
    mu = lax.broadcasted_iota(jnp.int32, (R, 1), 0).astype(jnp.float32) * (
        5.0 / (R - 1))
    rbf_t = jnp.exp(-GAMMA * (d - mu) ** 2)
    a = lax.dot_general(fW1_ref[...], rbf_t, (((0,), (0,)), ((), ())),
                        preferred_element_type=jnp.float32,
                        precision=lax.Precision.DEFAULT)
    z = _ssp(a + fb1_ref[...])
    w = lax.dot_general(z, fW2_ref[...], (((0,), (0,)), ((), ())),
                        preferred_element_type=jnp.float32,
                        precision=lax.Precision.DEFAULT)
    w_ref[...] = w + fb2_ref[...]


def _filter(s, fW1, fb1, fW2, fb2, hh):
    s3 = s.reshape(NBLK, 1, BE)
    return pl.pallas_call(
        _filter_body,
        grid=(NBLKH,),
        in_specs=[
            pl.BlockSpec((1, 1, BE), lambda i: (i + hh * NBLKH, 0, 0)),
            pl.BlockSpec((R, H), lambda i: (0, 0)),
            pl.BlockSpec((H, 1), lambda i: (0, 0)),
            pl.BlockSpec((H, H), lambda i: (0, 0)),
            pl.BlockSpec((1, H), lambda i: (0, 0)),
        ],
        out_specs=pl.BlockSpec((BE, H), lambda i: (i, 0)),
        out_shape=jax.ShapeDtypeStruct((EPADH, H), jnp.float32),
    )(s3, fW1, fb1.reshape(H, 1), fW2, fb2.reshape(1, H))


def _make_msg_body(hh):
  def _msg_body(w_hbm, h_hbm, ii_hbm, jj_hbm, z128_hbm,
              aggr_out,
              ibuf0, ibuf1, jbuf0, jbuf1, wbuf0, wbuf1, hbuf0, hbuf1, zbuf,
              aggr_sp,
              sii0, sii1, sjj0, sjj1, sw0, sw1, sg0, sg1, ss0, ss1):
    cid = lax.axis_index("c")
    sid = lax.axis_index("s")
    wid = cid * NS + sid
    r0 = sid * RPT

    ibuf = (ibuf0, ibuf1)
    jbuf = (jbuf0, jbuf1)
    wbuf = (wbuf0, wbuf1)
    hbuf = (hbuf0, hbuf1)
    sii = (sii0, sii1)
    sjj = (sjj0, sjj1)
    sw = (sw0, sw1)
    sg = (sg0, sg1)
    ss = (ss0, ss1)

    for tt in range(8):
        zr = 80 if tt < 7 else 72
        sl = pl.ds(r0 + tt * 80, zr)
        pltpu.sync_copy(z128_hbm.at[pl.ds(tt * 80, zr)], zbuf.at[pl.ds(0, zr)])
        pltpu.sync_copy(zbuf.at[pl.ds(0, zr)], aggr_sp.at[sl])
    plsc.subcore_barrier()

    base_l = wid * EPTH
    base_g = hh * EPADH + base_l

    def start_inputs(k, b):
        off_l = base_l + k * MCH
        off_g = base_g + k * MCH
        pltpu.async_copy(ii_hbm.at[pl.ds(off_g, MCH)], ibuf[b], sii[b])
        pltpu.async_copy(jj_hbm.at[pl.ds(off_g, MCH)], jbuf[b], sjj[b])
        pltpu.async_copy(w_hbm.at[pl.ds(off_l, MCH)], wbuf[b], sw[b])

    def wait_idx(k, b):
        off_g = base_g + k * MCH
        pltpu.make_async_copy(ii_hbm.at[pl.ds(off_g, MCH)], ibuf[b], sii[b]).wait()
        pltpu.make_async_copy(jj_hbm.at[pl.ds(off_g, MCH)], jbuf[b], sjj[b]).wait()

    def work(k, b, pref, swait):
        off = base_l + k * MCH
        if swait:
            pltpu.make_async_copy(
                wbuf[1 - b], aggr_sp.at[ibuf[1 - b]], ss[1 - b]).wait()
        if pref:
            start_inputs(k + 1, 1 - b)
        pltpu.make_async_copy(w_hbm.at[pl.ds(off, MCH)], wbuf[b], sw[b]).wait()
        pltpu.make_async_copy(h_hbm.at[jbuf[b]], hbuf[b], sg[b]).wait()
        if pref:
            wait_idx(k + 1, 1 - b)
            pltpu.async_copy(h_hbm.at[jbuf[1 - b]], hbuf[1 - b], sg[1 - b])

        def emul(e, c2):
            for c in range(H // L):
                sl = pl.ds(c * L, L)
                wbuf[b][e, sl] = wbuf[b][e, sl] * hbuf[b][e, sl]
            return c2

        lax.fori_loop(0, MCH, emul, 0)
        pltpu.async_copy(wbuf[b], aggr_sp.at[ibuf[b]], ss[b], add=True)

    start_inputs(0, 0)
    wait_idx(0, 0)
    pltpu.async_copy(h_hbm.at[jbuf[0]], hbuf[0], sg[0])
    work(0, 0, pref=True, swait=False)

    def pair(it, carry):
        k0 = 1 + it * 2
        work(k0, 1, True, True)
        work(k0 + 1, 0, True, True)
        return carry

    lax.fori_loop(0, (MNCHUNKH - 3) // 2, pair, 0)
    work(MNCHUNKH - 2, 1, pref=True, swait=True)
    work(MNCHUNKH - 1, 0, pref=False, swait=True)
    pltpu.make_async_copy(wbuf[0], aggr_sp.at[ibuf[0]], ss[0]).wait()
    plsc.subcore_barrier()

    for tt in range(8):
        zr = 80 if tt < 7 else 72
        sl = pl.ds(r0 + tt * 80, zr)
        pltpu.sync_copy(aggr_sp.at[sl], zbuf.at[pl.ds(0, zr)])
        pltpu.sync_copy(zbuf.at[pl.ds(0, zr)], aggr_out.at[cid, pl.ds(r0 + tt * 80, zr)])
  return _msg_body


def _msg(w, h, ii, jj, z128, hh):
    mesh = plsc.VectorSubcoreMesh(core_axis_name="c", subcore_axis_name="s",
                                  num_cores=NC, num_subcores=NS)
    return pl.kernel(
        _make_msg_body(hh),
        out_type=jax.ShapeDtypeStruct((NC, NPAD, H), jnp.float32),
        mesh=mesh,
        compiler_params=pltpu.CompilerParams(needs_layout_passes=False),
        scratch_types=(
            [pltpu.VMEM((MCH,), jnp.int32)] * 4
            + [pltpu.VMEM((MCH, H), jnp.float32)] * 4
            + [pltpu.VMEM((80, H), jnp.float32),
               pltpu.VMEM_SHARED((NPAD, H), jnp.float32)]
            + [pltpu.SemaphoreType.DMA] * 10
        ),
    )(w, h, ii, jj, z128)


def _update_body(aggr0_ref, aggr1_ref, deg_ref, uW1_ref, ub1_ref, uW2_ref,
                 ub2_ref, out_ref):
    a = (aggr0_ref[0] + aggr0_ref[1]) + (aggr1_ref[0] + aggr1_ref[1])
    dg = deg_ref[0] + deg_ref[1]
    a = a / jnp.maximum(dg, 1.0)
    z = _ssp(jnp.dot(a, uW1_ref[...],
                     preferred_element_type=jnp.float32) + ub1_ref[...])
    out_ref[...] = jnp.dot(z, uW2_ref[...],
                           preferred_element_type=jnp.float32) + ub2_ref[...]


def _update(aggr0, aggr1, deg, uW1, ub1, uW2, ub2):
    return pl.pallas_call(
        _update_body,
        grid=(N // BN,),
        in_specs=[
            pl.BlockSpec((NC, BN, H), lambda i: (0, i, 0)),
            pl.BlockSpec((NC, BN, H), lambda i: (0, i, 0)),
            pl.BlockSpec((NC, BN, 1), lambda i: (0, i, 0)),
            pl.BlockSpec((H, H), lambda i: (0, 0)),
            pl.BlockSpec((1, H), lambda i: (0, 0)),
            pl.BlockSpec((H, H), lambda i: (0, 0)),
            pl.BlockSpec((1, H), lambda i: (0, 0)),
        ],
        out_specs=pl.BlockSpec((BN, H), lambda i: (i, 0)),
        out_shape=jax.ShapeDtypeStruct((N, H), jnp.float32),
    )(aggr0, aggr1, deg, uW1, ub1.reshape(1, H), uW2, ub2.reshape(1, H))


def kernel(h, x, edge_index, fW1, fb1, fW2, fb2, uW1, ub1, uW2, ub2):
    ii = edge_index[0].astype(jnp.int32)
    jj = edge_index[1].astype(jnp.int32)
    npad = EPAD - E
    ii = jnp.concatenate([ii, jnp.full((npad,), DUMMY, jnp.int32)])
    jj = jnp.concatenate([jj, jnp.zeros((npad,), jnp.int32)])

    x8 = jnp.zeros((NPAD, 8), jnp.float32).at[:N, :3].set(x).reshape(NPAD * 8)
    z1 = jnp.zeros((NPAD,), jnp.float32)
    z128 = jnp.zeros((NPAD, H), jnp.float32)
    ones1 = jnp.ones((CH,), jnp.float32)

    s, deg = _geom(x8, ii, jj, ones1, z1)
    w0 = _filter(s, fW1, fb1, fW2, fb2, 0)
    aggr0 = _msg(w0, h, ii, jj, z128, 0)
    w1 = _filter(s, fW1, fb1, fW2, fb2, 1)
    aggr1 = _msg(w1, h, ii, jj, z128, 1)
    h_new = _update(aggr0, aggr1, deg.reshape(NC, NPAD, 1), uW1, ub1, uW2, ub2)
    return (h_new, x)
